# Initial kernel scaffold; baseline (speedup 1.0000x reference)
import jax
import jax.numpy as jnp
from jax.experimental import pallas as pl


def kernel(intensities, weight, bias, neighbors, level_deltas):
    raise NotImplementedError("write your pallas kernel here")


# speedup vs baseline: 10.6211x; 10.6211x over previous
import functools

import jax
import jax.numpy as jnp
from jax import lax
from jax.experimental import pallas as pl
from jax.experimental.pallas import tpu as pltpu
from jax.experimental.pallas import tpu_sc as plsc


def _sc_gather(nbr_t, table, k2, n, cin):
    info = plsc.get_sparse_core_info()
    nc, ns = info.num_cores, info.num_subcores
    nw = nc * ns
    per_w = n // nw
    assert per_w * nw == n
    chunk = 1000
    chunks = per_w // chunk
    assert chunks * chunk == per_w

    mesh = plsc.VectorSubcoreMesh(core_axis_name="c", subcore_axis_name="s")

    @functools.partial(
        pl.kernel,
        out_type=jax.ShapeDtypeStruct((k2, n, cin), jnp.float32),
        mesh=mesh,
        scratch_types=[
            pltpu.VMEM((k2 * chunk,), jnp.int32),
            pltpu.VMEM((k2, chunk, cin), jnp.float32),
            pltpu.SemaphoreType.DMA,
            pltpu.SemaphoreType.DMA,
        ],
        compiler_params=pltpu.CompilerParams(use_tc_tiling_on_sc=False),
    )
    def gather_kernel(nbr_hbm, tab_hbm, g_hbm, idx_v, gbuf, sem_i, sem_g):
        wid = lax.axis_index("s") * nc + lax.axis_index("c")
        base0 = wid * per_w

        def body(i, carry):
            base = base0 + i * chunk
            ics = [
                pltpu.async_copy(
                    nbr_hbm.at[pl.ds(k * n + base, chunk)],
                    idx_v.at[pl.ds(k * chunk, chunk)],
                    sem_i,
                )
                for k in range(k2)
            ]
            for cp in ics:
                cp.wait()
            gcs = [
                pltpu.async_copy(
                    tab_hbm.at[idx_v.at[pl.ds(k * chunk, chunk)]],
                    gbuf.at[k],
                    sem_g,
                )
                for k in range(k2)
            ]
            for cp in gcs:
                cp.wait()
            pltpu.sync_copy(gbuf, g_hbm.at[:, pl.ds(base, chunk), :])
            return carry

        lax.fori_loop(0, chunks, body, 0)

    return gather_kernel(nbr_t.reshape(-1), table)


def _tc_apply(g, wt, ld, bias2, n, k2, cin, nstencils, cout):
    bn = 800
    nb = n // bn
    assert nb * bn == n

    def body(g_ref, wt_ref, ld_ref, b_ref, o_ref):
        acc = jnp.zeros((bn, nstencils * cout), jnp.float32)
        for k in range(k2):
            acc = acc + jnp.dot(
                g_ref[k], wt_ref[k], preferred_element_type=jnp.float32
            )
        ld = ld_ref[...]
        out = jnp.zeros((bn, cout), jnp.float32)
        for s in range(nstencils):
            out = out + jnp.where(
                ld == s, acc[:, s * cout : (s + 1) * cout], 0.0
            )
        o_ref[...] = out + b_ref[...]

    return pl.pallas_call(
        body,
        grid=(nb,),
        in_specs=[
            pl.BlockSpec((k2, bn, cin), lambda i: (0, i, 0)),
            pl.BlockSpec((k2, cin, nstencils * cout), lambda i: (0, 0, 0)),
            pl.BlockSpec((bn, 1), lambda i: (i, 0)),
            pl.BlockSpec((1, cout), lambda i: (0, 0)),
        ],
        out_specs=pl.BlockSpec((bn, cout), lambda i: (i, 0)),
        out_shape=jax.ShapeDtypeStruct((n, cout), jnp.float32),
        compiler_params=pltpu.CompilerParams(
            dimension_semantics=("arbitrary",),
        ),
    )(g, wt, ld, bias2)


def kernel(intensities, weight, bias, neighbors, level_deltas):
    b, cin, n = intensities.shape
    cout, _, nstencils, kh, kw = weight.shape
    k2 = kh * kw

    table = intensities[0].T
    nbr_t = neighbors.astype(jnp.int32).T
    ld = level_deltas.astype(jnp.int32).reshape(n, 1)
    wt = jnp.transpose(weight, (3, 4, 1, 2, 0)).reshape(
        k2, cin, nstencils * cout
    )
    bias2 = bias.reshape(1, cout)

    g = _sc_gather(nbr_t, table, k2, n, cin)
    out2 = _tc_apply(g, wt, ld, bias2, n, k2, cin, nstencils, cout)
    return out2.T.reshape(b, cout, n)
